# Initial kernel scaffold; baseline (speedup 1.0000x reference)
import jax
import jax.numpy as jnp
from jax.experimental import pallas as pl


def kernel(input_seq, hidden, masks, logodds):
    raise NotImplementedError("write your pallas kernel here")


# speedup vs baseline: 69.0969x; 69.0969x over previous
import jax
import jax.numpy as jnp
from jax import lax
from jax.experimental import pallas as pl
from jax.experimental.pallas import tpu as pltpu
from jax.experimental.pallas import tpu_sc as plsc

VOCAB = 100000
BATCH = 4096
SEQ = 200

NC = 2
NS = 16
L = 16
NW = NC * NS
ROWS_PER_W = BATCH // NW
SUB = 32
NSUB = ROWS_PER_W // SUB
BLK = SEQ * SUB


def _sc_kernel(idx_hbm, mask_hbm, table_hbm, out_hbm, table_v, idx_v, mask_v, vals_v):
    wid = lax.axis_index("s") * NC + lax.axis_index("c")
    pltpu.sync_copy(table_hbm, table_v)

    for sb in range(NSUB):
        off = (wid * NSUB + sb) * BLK
        pltpu.sync_copy(idx_hbm.at[pl.ds(off, BLK)], idx_v)
        pltpu.sync_copy(mask_hbm.at[pl.ds(off, BLK)], mask_v)
        for g in range(SUB // L):
            col = g * L

            def pass1(j, rmax):
                iv = idx_v[pl.ds(j * SUB + col, L)]
                gv = plsc.load_gather(table_v, [iv])
                mv = mask_v[pl.ds(j * SUB + col, L)]
                val = jnp.where(mv != 0, -jnp.inf, gv)
                vals_v[pl.ds(j * SUB + col, L)] = val
                return jnp.maximum(rmax, val)

            rmax = lax.fori_loop(
                0, SEQ, pass1, jnp.full((L,), -jnp.inf, jnp.float32))

            def pass2(j, acc):
                e = jnp.exp(vals_v[pl.ds(j * SUB + col, L)] - rmax)
                vals_v[pl.ds(j * SUB + col, L)] = e
                return acc + e

            ssum = lax.fori_loop(
                0, SEQ, pass2, jnp.zeros((L,), jnp.float32))
            inv = 1.0 / ssum

            def pass3(j, carry):
                vals_v[pl.ds(j * SUB + col, L)] = vals_v[pl.ds(j * SUB + col, L)] * inv
                return carry

            lax.fori_loop(0, SEQ, pass3, 0)
        pltpu.sync_copy(vals_v, out_hbm.at[pl.ds(off, BLK)])


@jax.jit
def _log_odds_attention(idx_flat, mask_flat, logodds):
    mesh = plsc.VectorSubcoreMesh(core_axis_name="c", subcore_axis_name="s")
    return pl.kernel(
        _sc_kernel,
        mesh=mesh,
        compiler_params=pltpu.CompilerParams(needs_layout_passes=False),
        out_type=jax.ShapeDtypeStruct((BATCH * SEQ,), jnp.float32),
        scratch_types=[
            pltpu.VMEM((VOCAB,), jnp.float32),
            pltpu.VMEM((BLK,), jnp.int32),
            pltpu.VMEM((BLK,), jnp.int32),
            pltpu.VMEM((BLK,), jnp.float32),
        ],
    )(idx_flat, mask_flat, logodds)


def _to_flat(x):
    return x.reshape(NW * NSUB, SUB, SEQ).transpose(0, 2, 1).reshape(-1)


def kernel(input_seq, hidden, masks, logodds):
    del hidden
    idx_flat = _to_flat(input_seq.astype(jnp.int32))
    mask_flat = _to_flat(masks.astype(jnp.int32))
    out_flat = _log_odds_attention(idx_flat, mask_flat, logodds)
    return out_flat.reshape(NW * NSUB, SEQ, SUB).transpose(0, 2, 1).reshape(BATCH, SEQ)


# speedup vs baseline: 142.9623x; 2.0690x over previous
import jax
import jax.numpy as jnp
from jax import lax
from jax.experimental import pallas as pl
from jax.experimental.pallas import tpu as pltpu
from jax.experimental.pallas import tpu_sc as plsc

VOCAB = 100000
BATCH = 4096
SEQ = 200

NC = 2
NS = 16
L = 16
NW = NC * NS
ROWS_PER_W = BATCH // NW
SUB = 32
NSUB = ROWS_PER_W // SUB
BLK = SUB * SEQ

SENT = VOCAB
TPAD = VOCAB + L


def _sc_kernel(idx_hbm, table_hbm, out_hbm, table_v, idx_v, vals_v, out_v):
    wid = lax.axis_index("s") * NC + lax.axis_index("c")
    pltpu.sync_copy(table_hbm, table_v.at[pl.ds(0, VOCAB)])
    table_v[pl.ds(VOCAB, L)] = jnp.full((L,), -jnp.inf, jnp.float32)
    lane_off = lax.iota(jnp.int32, L) * SEQ

    for sb in range(NSUB):
        off = (wid * NSUB + sb) * BLK
        pltpu.sync_copy(idx_hbm.at[pl.ds(off, BLK)], idx_v)
        for g in range(SUB // L):
            col = g * L
            base_vec = lane_off + (col * SEQ)

            @plsc.parallel_loop(
                0, SEQ, unroll=8,
                carry=jnp.full((L,), -jnp.inf, jnp.float32))
            def rmax(j, m):
                pos = base_vec + j
                iv = plsc.load_gather(idx_v, [pos])
                gv = plsc.load_gather(table_v, [iv])
                vals_v[pl.ds(j * SUB + col, L)] = gv
                return jnp.maximum(m, gv)

            @plsc.parallel_loop(
                0, SEQ, unroll=8, carry=jnp.zeros((L,), jnp.float32))
            def ssum(j, acc):
                e = jnp.exp(vals_v[pl.ds(j * SUB + col, L)] - rmax)
                vals_v[pl.ds(j * SUB + col, L)] = e
                return acc + e

            inv = 1.0 / ssum

            @plsc.parallel_loop(0, SEQ, unroll=8)
            def _rescale(j):
                val = vals_v[pl.ds(j * SUB + col, L)] * inv
                plsc.store_scatter(out_v, [base_vec + j], val)

        pltpu.sync_copy(out_v, out_hbm.at[pl.ds(off, BLK)])


@jax.jit
def _log_odds_attention(idx_flat, logodds):
    mesh = plsc.VectorSubcoreMesh(core_axis_name="c", subcore_axis_name="s")
    return pl.kernel(
        _sc_kernel,
        mesh=mesh,
        compiler_params=pltpu.CompilerParams(needs_layout_passes=False),
        out_type=jax.ShapeDtypeStruct((BATCH * SEQ,), jnp.float32),
        scratch_types=[
            pltpu.VMEM((TPAD,), jnp.float32),
            pltpu.VMEM((BLK,), jnp.int32),
            pltpu.VMEM((BLK,), jnp.float32),
            pltpu.VMEM((BLK,), jnp.float32),
        ],
    )(idx_flat, logodds)


def kernel(input_seq, hidden, masks, logodds):
    del hidden
    idx_flat = jnp.where(masks, SENT, input_seq.astype(jnp.int32)).reshape(-1)
    out_flat = _log_odds_attention(idx_flat, logodds)
    return out_flat.reshape(BATCH, SEQ)
